# baseline (device time: 63089 ns/iter reference)
import jax
import jax.numpy as jnp
from jax import lax
from jax.experimental import pallas as pl
from jax.experimental.pallas import tpu as pltpu

N_DEV = 4


def kernel(x, Wg, Wu, Wd):
    m, _ = x.shape
    d = Wd.shape[1]

    def body(x_ref, wg_ref, wu_ref, wd_ref, out_ref, comm_ref,
             send_sems, recv_sems):
        my = lax.axis_index("i")
        left = (my - 1) % N_DEV
        right = (my + 1) % N_DEV

        barrier_sem = pltpu.get_barrier_semaphore()
        for nbr in (left, right):
            pl.semaphore_signal(
                barrier_sem, inc=1,
                device_id=(nbr,), device_id_type=pl.DeviceIdType.MESH,
            )
        pl.semaphore_wait(barrier_sem, 2)

        xb = x_ref[...].astype(jnp.bfloat16)
        gate = jnp.dot(xb, wg_ref[...].astype(jnp.bfloat16),
                       preferred_element_type=jnp.float32)
        up = jnp.dot(xb, wu_ref[...].astype(jnp.bfloat16),
                     preferred_element_type=jnp.float32)
        hidden = (gate * (up * jax.nn.sigmoid(up))).astype(jnp.bfloat16)
        partial = jnp.dot(hidden, wd_ref[...].astype(jnp.bfloat16),
                          preferred_element_type=jnp.float32)

        comm_ref[0] = partial.astype(jnp.bfloat16)
        acc = partial
        for h in range(N_DEV - 1):
            rdma = pltpu.make_async_remote_copy(
                src_ref=comm_ref.at[h],
                dst_ref=comm_ref.at[h + 1],
                send_sem=send_sems.at[h],
                recv_sem=recv_sems.at[h],
                device_id=(right,),
                device_id_type=pl.DeviceIdType.MESH,
            )
            rdma.start()
            rdma.wait()
            acc = acc + comm_ref[h + 1].astype(jnp.float32)
        out_ref[...] = acc

    return pl.pallas_call(
        body,
        out_shape=jax.ShapeDtypeStruct((m, d), jnp.float32),
        in_specs=[pl.BlockSpec(memory_space=pltpu.VMEM)] * 4,
        out_specs=pl.BlockSpec(memory_space=pltpu.VMEM),
        scratch_shapes=[
            pltpu.VMEM((N_DEV, m, d), jnp.bfloat16),
            pltpu.SemaphoreType.DMA((N_DEV - 1,)),
            pltpu.SemaphoreType.DMA((N_DEV - 1,)),
        ],
        compiler_params=pltpu.CompilerParams(collective_id=0),
    )(x, Wg, Wu, Wd)


# device time: 32586 ns/iter; 1.9361x vs baseline; 1.9361x over previous
import jax
import jax.numpy as jnp
from jax import lax
from jax.experimental import pallas as pl
from jax.experimental.pallas import tpu as pltpu

N_DEV = 4
CHUNK = 192


def kernel(x, Wg, Wu, Wd):
    m, _ = x.shape
    d_out = Wd.shape[1]

    def body(x_ref, wg_ref, wu_ref, wd_ref, out_ref,
             send_buf, rs_recv, ag_src, ag_recv,
             rs_send_sems, rs_recv_sems, ag_send_sems, ag_recv_sems):
        my = lax.axis_index("i")

        barrier_sem = pltpu.get_barrier_semaphore()
        for k in (1, 2, 3):
            pl.semaphore_signal(
                barrier_sem, inc=1,
                device_id=((my + k) % N_DEV,),
                device_id_type=pl.DeviceIdType.MESH,
            )
        pl.semaphore_wait(barrier_sem, 3)

        wgb = wg_ref[...].astype(jnp.bfloat16)
        wub = wu_ref[...].astype(jnp.bfloat16)
        wdb = wd_ref[...].astype(jnp.bfloat16)

        def partial_chunk(row_start):
            xb = x_ref[pl.ds(row_start, CHUNK), :].astype(jnp.bfloat16)
            gate = jnp.dot(xb, wgb, preferred_element_type=jnp.float32)
            up = jnp.dot(xb, wub, preferred_element_type=jnp.float32)
            hidden = (gate * (up * jax.nn.sigmoid(up))).astype(jnp.bfloat16)
            return jnp.dot(hidden, wdb, preferred_element_type=jnp.float32)

        rs_rdmas = []
        for k in (1, 2, 3):
            tgt = (my + k) % N_DEV
            send_buf[k - 1] = partial_chunk(tgt * CHUNK).astype(jnp.bfloat16)
            rdma = pltpu.make_async_remote_copy(
                src_ref=send_buf.at[k - 1],
                dst_ref=rs_recv.at[k - 1],
                send_sem=rs_send_sems.at[k - 1],
                recv_sem=rs_recv_sems.at[k - 1],
                device_id=(tgt,),
                device_id_type=pl.DeviceIdType.MESH,
            )
            rdma.start()
            rs_rdmas.append(rdma)

        acc = partial_chunk(my * CHUNK)
        for k in (1, 2, 3):
            rs_rdmas[k - 1].wait_recv()
            acc = acc + rs_recv[k - 1].astype(jnp.float32)

        red = acc.astype(jnp.bfloat16)
        ag_src[...] = red
        out_ref[pl.ds(my * CHUNK, CHUNK), :] = red
        ag_rdmas = []
        for k in (1, 2, 3):
            tgt = (my + k) % N_DEV
            rdma = pltpu.make_async_remote_copy(
                src_ref=ag_src,
                dst_ref=ag_recv.at[k - 1],
                send_sem=ag_send_sems.at[k - 1],
                recv_sem=ag_recv_sems.at[k - 1],
                device_id=(tgt,),
                device_id_type=pl.DeviceIdType.MESH,
            )
            rdma.start()
            ag_rdmas.append(rdma)

        for k in (1, 2, 3):
            ag_rdmas[k - 1].wait_recv()
            src = (my - k) % N_DEV
            out_ref[pl.ds(src * CHUNK, CHUNK), :] = ag_recv[k - 1]

        for k in (1, 2, 3):
            rs_rdmas[k - 1].wait_send()
            ag_rdmas[k - 1].wait_send()

    return pl.pallas_call(
        body,
        out_shape=jax.ShapeDtypeStruct((m, d_out), jnp.bfloat16),
        in_specs=[pl.BlockSpec(memory_space=pltpu.VMEM)] * 4,
        out_specs=pl.BlockSpec(memory_space=pltpu.VMEM),
        scratch_shapes=[
            pltpu.VMEM((N_DEV - 1, CHUNK, d_out), jnp.bfloat16),
            pltpu.VMEM((N_DEV - 1, CHUNK, d_out), jnp.bfloat16),
            pltpu.VMEM((CHUNK, d_out), jnp.bfloat16),
            pltpu.VMEM((N_DEV - 1, CHUNK, d_out), jnp.bfloat16),
            pltpu.SemaphoreType.DMA((N_DEV - 1,)),
            pltpu.SemaphoreType.DMA((N_DEV - 1,)),
            pltpu.SemaphoreType.DMA((N_DEV - 1,)),
            pltpu.SemaphoreType.DMA((N_DEV - 1,)),
        ],
        compiler_params=pltpu.CompilerParams(collective_id=0),
    )(x, Wg, Wu, Wd)


# device time: 31632 ns/iter; 1.9945x vs baseline; 1.0302x over previous
import jax
import jax.numpy as jnp
from jax import lax
from jax.experimental import pallas as pl
from jax.experimental.pallas import tpu as pltpu

N_DEV = 4
CHUNK = 192


def kernel(x, Wg, Wu, Wd):
    m, _ = x.shape
    d_out = Wd.shape[1]

    def body(x_ref, wg_ref, wu_ref, wd_ref, out_ref,
             send_buf, rs_recv, ag_src, ag_recv,
             rs_send_sems, rs_recv_sems, ag_send_sems, ag_recv_sems):
        my = lax.axis_index("i")

        barrier_sem = pltpu.get_barrier_semaphore()
        for k in (1, 2, 3):
            pl.semaphore_signal(
                barrier_sem, inc=1,
                device_id=((my + k) % N_DEV,),
                device_id_type=pl.DeviceIdType.MESH,
            )
        pl.semaphore_wait(barrier_sem, 3)

        wgb = wg_ref[...].astype(jnp.bfloat16)
        wub = wu_ref[...].astype(jnp.bfloat16)
        wdb = wd_ref[...].astype(jnp.bfloat16)

        def gate_up(row_start):
            xb = x_ref[pl.ds(row_start, CHUNK), :].astype(jnp.bfloat16)
            gate = jnp.dot(xb, wgb, preferred_element_type=jnp.float32)
            up = jnp.dot(xb, wub, preferred_element_type=jnp.float32)
            return (gate * (up * jax.nn.sigmoid(up))).astype(jnp.bfloat16)

        def partial_chunk(row_start):
            return jnp.dot(gate_up(row_start), wdb,
                           preferred_element_type=jnp.float32)

        rs_rdmas = []
        for k in (1, 2):
            tgt = (my + k) % N_DEV
            send_buf[k - 1] = partial_chunk(tgt * CHUNK).astype(jnp.bfloat16)
            rdma = pltpu.make_async_remote_copy(
                src_ref=send_buf.at[k - 1],
                dst_ref=rs_recv.at[k - 1],
                send_sem=rs_send_sems.at[k - 1],
                recv_sem=rs_recv_sems.at[k - 1],
                device_id=(tgt,),
                device_id_type=pl.DeviceIdType.MESH,
            )
            rdma.start()
            rs_rdmas.append(rdma)

        tgt3 = (my + 3) % N_DEV
        hidden3 = gate_up(tgt3 * CHUNK)
        half = d_out // 2
        for h in (0, 1):
            lo, hi = h * half, (h + 1) * half
            send_buf[2, :, lo:hi] = jnp.dot(
                hidden3, wdb[:, lo:hi],
                preferred_element_type=jnp.float32).astype(jnp.bfloat16)
            rdma = pltpu.make_async_remote_copy(
                src_ref=send_buf.at[2, :, lo:hi],
                dst_ref=rs_recv.at[2, :, lo:hi],
                send_sem=rs_send_sems.at[2 + h],
                recv_sem=rs_recv_sems.at[2 + h],
                device_id=(tgt3,),
                device_id_type=pl.DeviceIdType.MESH,
            )
            rdma.start()
            rs_rdmas.append(rdma)

        own = partial_chunk(my * CHUNK)

        ag_rdmas = []
        for h in (0, 1):
            lo, hi = h * half, (h + 1) * half
            if h == 0:
                for r in (rs_rdmas[0], rs_rdmas[1], rs_rdmas[2]):
                    r.wait_recv()
            else:
                rs_rdmas[3].wait_recv()
            acc = own[:, lo:hi]
            for s in (0, 1, 2):
                acc = acc + rs_recv[s, :, lo:hi].astype(jnp.float32)
            red = acc.astype(jnp.bfloat16)
            ag_src[:, lo:hi] = red
            out_ref[pl.ds(my * CHUNK, CHUNK), lo:hi] = red
            for k in (1, 2, 3):
                tgt = (my + k) % N_DEV
                rdma = pltpu.make_async_remote_copy(
                    src_ref=ag_src.at[:, lo:hi],
                    dst_ref=ag_recv.at[k - 1, :, lo:hi],
                    send_sem=ag_send_sems.at[h * 3 + k - 1],
                    recv_sem=ag_recv_sems.at[h * 3 + k - 1],
                    device_id=(tgt,),
                    device_id_type=pl.DeviceIdType.MESH,
                )
                rdma.start()
                ag_rdmas.append(rdma)

        for k in (1, 2, 3):
            ag_rdmas[k - 1].wait_recv()
            ag_rdmas[3 + k - 1].wait_recv()
            src = (my - k) % N_DEV
            out_ref[pl.ds(src * CHUNK, CHUNK), :] = ag_recv[k - 1]

        for r in rs_rdmas:
            r.wait_send()
        for r in ag_rdmas:
            r.wait_send()

    return pl.pallas_call(
        body,
        out_shape=jax.ShapeDtypeStruct((m, d_out), jnp.bfloat16),
        in_specs=[pl.BlockSpec(memory_space=pltpu.VMEM)] * 4,
        out_specs=pl.BlockSpec(memory_space=pltpu.VMEM),
        scratch_shapes=[
            pltpu.VMEM((N_DEV - 1, CHUNK, d_out), jnp.bfloat16),
            pltpu.VMEM((N_DEV - 1, CHUNK, d_out), jnp.bfloat16),
            pltpu.VMEM((CHUNK, d_out), jnp.bfloat16),
            pltpu.VMEM((N_DEV - 1, CHUNK, d_out), jnp.bfloat16),
            pltpu.SemaphoreType.DMA((N_DEV,)),
            pltpu.SemaphoreType.DMA((N_DEV,)),
            pltpu.SemaphoreType.DMA((6,)),
            pltpu.SemaphoreType.DMA((6,)),
        ],
        compiler_params=pltpu.CompilerParams(collective_id=0),
    )(x, Wg, Wu, Wd)
